# baseline (speedup 1.0000x reference)
import jax
import jax.numpy as jnp
from jax.experimental import pallas as pl
from jax.experimental.pallas import tpu as pltpu

_B, _H, _W = 16, 384, 384
_C = 32
_HW = _H * _W
_ROWS = _HW // 128
_K_TARGET = _HW - int(round(_HW * 0.1))


def _fused_kernel(vals_ref, out_ref, mask_ref, sem):
    b = pl.program_id(0)

    x = vals_ref[0]
    bits = jax.lax.bitcast_convert_type(x, jnp.int32)
    ikey = jnp.where(bits >= 0, bits, bits ^ jnp.int32(0x7FFFFFFF))

    def body(j, k):
        trial = k + (jnp.int32(1) << (jnp.int32(31) - j))
        cnt = jnp.sum((ikey < trial).astype(jnp.int32))
        return jnp.where(cnt < _K_TARGET, trial, k)

    k = jax.lax.fori_loop(0, 32, body, jnp.int32(jnp.iinfo(jnp.int32).min))
    dbits = jnp.where(k >= 0, k, k ^ jnp.int32(0x7FFFFFFF))
    d = jax.lax.bitcast_convert_type(dbits, jnp.float32)

    sel = jax.lax.rem(b, 2)

    @pl.when(b >= 2)
    def _():
        for c in range(_C):
            pltpu.make_async_copy(
                mask_ref.at[sel], out_ref.at[b - 2, c], sem).wait()

    mask_ref[sel] = (x > d).astype(jnp.float32)
    for c in range(_C):
        pltpu.make_async_copy(mask_ref.at[sel], out_ref.at[b, c], sem).start()

    @pl.when(b == _B - 1)
    def _():
        for bb in (_B - 2, _B - 1):
            for c in range(_C):
                pltpu.make_async_copy(
                    mask_ref.at[jax.lax.rem(jnp.int32(bb), 2)],
                    out_ref.at[bb, c], sem).wait()


@jax.jit
def kernel(vals):
    vals3 = vals.reshape(_B, _ROWS, 128)
    out = pl.pallas_call(
        _fused_kernel,
        grid=(_B,),
        in_specs=[pl.BlockSpec((1, _ROWS, 128), lambda b: (b, 0, 0))],
        out_specs=pl.BlockSpec(memory_space=pl.ANY),
        out_shape=jax.ShapeDtypeStruct((_B, _C, _ROWS, 128), jnp.float32),
        scratch_shapes=[
            pltpu.VMEM((2, _ROWS, 128), jnp.float32),
            pltpu.SemaphoreType.DMA,
        ],
    )(vals3)
    return out.reshape(_B, _C, _H, _W)


# speedup vs baseline: 3.1102x; 3.1102x over previous
import jax
import jax.numpy as jnp
from jax.experimental import pallas as pl
from jax.experimental.pallas import tpu as pltpu

_B, _H, _W = 16, 384, 384
_C = 32
_HW = _H * _W
_K_TARGET = _HW - int(round(_HW * 0.1))


def _fused_kernel(vals_ref, out_ref, mask_ref, sem):
    b = pl.program_id(0)

    x = vals_ref[0]
    bits = jax.lax.bitcast_convert_type(x, jnp.int32)
    ikey = jnp.where(bits >= 0, bits, bits ^ jnp.int32(0x7FFFFFFF))

    def body(j, k):
        trial = k + (jnp.int32(1) << (jnp.int32(31) - j))
        cnt = jnp.sum((ikey < trial).astype(jnp.int32))
        return jnp.where(cnt < _K_TARGET, trial, k)

    k = jax.lax.fori_loop(0, 32, body, jnp.int32(jnp.iinfo(jnp.int32).min))
    dbits = jnp.where(k >= 0, k, k ^ jnp.int32(0x7FFFFFFF))
    d = jax.lax.bitcast_convert_type(dbits, jnp.float32)

    sel = jax.lax.rem(b, 2)

    @pl.when(b >= 2)
    def _():
        for c in range(_C):
            pltpu.make_async_copy(
                mask_ref.at[sel], out_ref.at[b - 2, c], sem).wait()

    mask_ref[sel] = (x > d).astype(jnp.float32)
    for c in range(_C):
        pltpu.make_async_copy(mask_ref.at[sel], out_ref.at[b, c], sem).start()

    @pl.when(b == _B - 1)
    def _():
        for bb in (_B - 2, _B - 1):
            for c in range(_C):
                pltpu.make_async_copy(
                    mask_ref.at[jax.lax.rem(jnp.int32(bb), 2)],
                    out_ref.at[bb, c], sem).wait()


@jax.jit
def kernel(vals):
    out = pl.pallas_call(
        _fused_kernel,
        grid=(_B,),
        in_specs=[pl.BlockSpec((1, _H, _W), lambda b: (b, 0, 0))],
        out_specs=pl.BlockSpec(memory_space=pl.ANY),
        out_shape=jax.ShapeDtypeStruct((_B, _C, _H, _W), jnp.float32),
        scratch_shapes=[
            pltpu.VMEM((2, _H, _W), jnp.float32),
            pltpu.SemaphoreType.DMA,
        ],
    )(vals)
    return out
